# baseline (speedup 1.0000x reference)
import functools

import jax
import jax.numpy as jnp
from jax import lax
from jax.experimental import pallas as pl
from jax.experimental.pallas import tpu as pltpu
from jax.experimental.pallas import tpu_sc as plsc

F = 20
B = 1024
NPROD = 100000
L = 16


def _sc_info():
  try:
    info = plsc.get_sparse_core_info()
    return info.num_cores, info.num_subcores
  except Exception:
    return 2, 16


def _sc_gather(user, product, uf_flat, pf_flat):
  nc, ns = _sc_info()
  nw = nc * ns
  bpw = B // nw

  mesh = plsc.VectorSubcoreMesh(core_axis_name="c", subcore_axis_name="s")

  @functools.partial(
      pl.kernel,
      mesh=mesh,
      compiler_params=pltpu.CompilerParams(use_tc_tiling_on_sc=False),
      out_type=[
          jax.ShapeDtypeStruct((F, B), jnp.float32),
          jax.ShapeDtypeStruct((F, B), jnp.float32),
      ],
      scratch_types=[
          pltpu.VMEM((bpw,), jnp.int32),
          pltpu.VMEM((bpw,), jnp.int32),
          pltpu.VMEM((F, bpw), jnp.int32),
          pltpu.VMEM((F, bpw), jnp.int32),
          pltpu.VMEM((F, bpw), jnp.float32),
          pltpu.VMEM((F, bpw), jnp.float32),
          pltpu.SemaphoreType.DMA,
          pltpu.SemaphoreType.DMA,
      ],
  )
  def gather_kernel(user_hbm, prod_hbm, uf_hbm, pf_hbm, u_out, p_out,
                    uidx_v, pidx_v, uflat_v, pflat_v, urows_v, pcols_v,
                    sem_u, sem_p):
    wid = lax.axis_index("s") * nc + lax.axis_index("c")
    base = wid * bpw
    pltpu.sync_copy(user_hbm.at[pl.ds(base, bpw)], uidx_v)
    pltpu.sync_copy(prod_hbm.at[pl.ds(base, bpw)], pidx_v)
    uparts = [uidx_v[pl.ds(i * L, L)] * F for i in range(bpw // L)]
    pparts = [pidx_v[pl.ds(i * L, L)] for i in range(bpw // L)]
    for k in range(F):
      for i in range(bpw // L):
        uflat_v[k, pl.ds(i * L, L)] = uparts[i] + k
        pflat_v[k, pl.ds(i * L, L)] = pparts[i] + k * NPROD
    copies = [
        pltpu.async_copy(uf_hbm.at[uflat_v.at[k]], urows_v.at[k], sem_u)
        for k in range(F)
    ] + [
        pltpu.async_copy(pf_hbm.at[pflat_v.at[k]], pcols_v.at[k], sem_p)
        for k in range(F)
    ]
    for c in copies:
      c.wait()
    pltpu.sync_copy(urows_v, u_out.at[:, pl.ds(base, bpw)])
    pltpu.sync_copy(pcols_v, p_out.at[:, pl.ds(base, bpw)])

  return gather_kernel(user, product, uf_flat, pf_flat)


def _mm_body(ut_ref, p_ref, o_ref):
  o_ref[...] = lax.dot_general(
      ut_ref[...], p_ref[...], (((0,), (0,)), ((), ())),
      preferred_element_type=jnp.float32)


def _tc_matmul(ut, p):
  bm = 256
  return pl.pallas_call(
      _mm_body,
      grid=(B // bm,),
      in_specs=[
          pl.BlockSpec((F, bm), lambda i: (0, i)),
          pl.BlockSpec((F, B), lambda i: (0, 0)),
      ],
      out_specs=pl.BlockSpec((bm, B), lambda i: (i, 0)),
      out_shape=jax.ShapeDtypeStruct((B, B), jnp.float32),
  )(ut, p)


def kernel(user, product, user_factors, product_factors):
  ut, p = _sc_gather(user, product, user_factors.reshape(-1),
                     product_factors.reshape(-1))
  return _tc_matmul(ut, p)


# speedup vs baseline: 21.9655x; 21.9655x over previous
import functools

import jax
import jax.numpy as jnp
from jax import lax
from jax.experimental import pallas as pl
from jax.experimental.pallas import tpu as pltpu
from jax.experimental.pallas import tpu_sc as plsc

F = 20
K = 32
B = 1024
L = 16
NC, NS = 2, 16
NW = NC * NS
BPW = B // NW
WS = 24


def _sc_gather(user, product, uf_t, pf):
  mesh = plsc.VectorSubcoreMesh(core_axis_name="c", subcore_axis_name="s")

  @functools.partial(
      pl.kernel,
      mesh=mesh,
      compiler_params=pltpu.CompilerParams(
          use_tc_tiling_on_sc=True, needs_layout_passes=False),
      out_type=[
          jax.ShapeDtypeStruct((B, K), jnp.float32),
          jax.ShapeDtypeStruct((B, K), jnp.float32),
      ],
      scratch_types=[
          pltpu.VMEM((BPW,), jnp.int32),
          pltpu.VMEM((BPW,), jnp.int32),
          pltpu.VMEM((BPW * WS, 128), jnp.float32),
          pltpu.VMEM((BPW, K), jnp.float32),
          pltpu.VMEM((BPW, K), jnp.float32),
          pltpu.SemaphoreType.DMA,
      ],
  )
  def gather_kernel(user_hbm, prod_hbm, uft_hbm, pf_hbm, u_out, p_out,
                    uidx_v, pidx_v, win_v, u_slab, p_slab, sem):
    c = lax.axis_index("c")
    s = lax.axis_index("s")
    base = (c * NS + s) * BPW
    pltpu.sync_copy(user_hbm.at[pl.ds(base, BPW)], uidx_v)
    pltpu.sync_copy(prod_hbm.at[pl.ds(base, BPW)], pidx_v)

    iota = lax.iota(jnp.int32, L)
    zeros = jnp.zeros((L,), jnp.float32)
    pad_mask = iota < (F - 8)

    def gather_one(idx_v, table_hbm, slab_v):
      copies = []
      for i in range(BPW // L):
        part = lax.bitwise_and(idx_v[pl.ds(i * L, L)], -128)
        for j in range(L):
          b = i * L + j
          col = pl.multiple_of(part[j], 128)
          copies.append(pltpu.async_copy(
              table_hbm.at[:, pl.ds(col, 128)],
              win_v.at[pl.ds(b * WS, F)], sem))
      for cp in copies:
        cp.wait()
      for i in range(BPW // L):
        lo_part = lax.bitwise_and(idx_v[pl.ds(i * L, L)], 127)
        for j in range(L):
          b = i * L + j
          lo = jnp.full((L,), lo_part[j], jnp.int32)
          v1 = plsc.load_gather(win_v, [b * WS + iota, lo])
          v2 = plsc.load_gather(win_v, [b * WS + 8 + iota, lo])
          slab_v[b, pl.ds(L, L)] = zeros
          slab_v[b, pl.ds(8, L)] = jnp.where(pad_mask, v2, 0.0)
          slab_v[b, pl.ds(0, L)] = v1

    gather_one(uidx_v, uft_hbm, u_slab)
    pltpu.sync_copy(u_slab, u_out.at[pl.ds(base, BPW)])
    gather_one(pidx_v, pf_hbm, p_slab)
    pltpu.sync_copy(p_slab, p_out.at[pl.ds(base, BPW)])

  return gather_kernel(user, product, uf_t, pf)


def _mm_body(u_ref, p_ref, o_ref):
  o_ref[...] = lax.dot_general(
      u_ref[...], p_ref[...], (((1,), (1,)), ((), ())),
      preferred_element_type=jnp.float32)


def _tc_matmul(u, p):
  bm = 256
  return pl.pallas_call(
      _mm_body,
      grid=(B // bm,),
      in_specs=[
          pl.BlockSpec((bm, K), lambda i: (i, 0)),
          pl.BlockSpec((B, K), lambda i: (0, 0)),
      ],
      out_specs=pl.BlockSpec((bm, B), lambda i: (i, 0)),
      out_shape=jax.ShapeDtypeStruct((B, B), jnp.float32),
  )(u, p)


def kernel(user, product, user_factors, product_factors):
  u, p = _sc_gather(user, product, user_factors.T, product_factors)
  return _tc_matmul(u, p)
